# Initial kernel scaffold; baseline (speedup 1.0000x reference)
import jax
import jax.numpy as jnp
from jax.experimental import pallas as pl


def kernel(polar_feat, ref_feat):
    raise NotImplementedError("write your pallas kernel here")


# speedup vs baseline: 15.5021x; 15.5021x over previous
import functools

import jax
import jax.numpy as jnp
import numpy as np
from jax import lax
from jax.experimental import pallas as pl
from jax.experimental.pallas import tpu as pltpu
from jax.experimental.pallas import tpu_sc as plsc

_POLAR = (64, 1024)
_CART = (512, 512)
_CDGS = 3.0
_B, _C = 4, 32

_PLANES = _B * _C
_NPX = _CART[0] * _CART[1]
_PLANE_W = _POLAR[0] * _POLAR[1]
_BOX_R0, _BOX_C0, _BOX_H, _BOX_W = 240, 240, 32, 32
_BOX = _BOX_H * _BOX_W
_PAD = 1032
_TW = _PLANE_W + _BOX + _PAD

_K = 4096
_NCHUNK = _NPX // _K
_NTILES = 32
_PPT = _PLANES // _NTILES


def _build_static():
    yy_org, xx_org = np.meshgrid(np.arange(_CART[0]), np.arange(_CART[1]),
                                 indexing='ij')
    yy = (yy_org - _CART[0] / 2.0 + 0.5).astype(np.float32)
    xx = (xx_org - _CART[1] / 2.0 + 0.5).astype(np.float32)
    depth = np.sqrt(xx ** 2 + yy ** 2)
    phi = np.pi - np.arctan2(yy, xx)
    index_y = depth / (_CART[0] / 2.0 * np.sqrt(2.0)) * (_POLAR[0] + _CDGS) - _CDGS
    index_x = phi / np.pi / 2.0 * _POLAR[1]
    mask = index_y > 0
    gx = (index_x / _POLAR[1] * 2.0 - 1.0).astype(np.float32)
    gy = (-(index_y / _POLAR[0] * 2.0 - 1.0)).astype(np.float32)
    ix = ((gx + np.float32(1.0)) * np.float32(0.5) * np.float32(_POLAR[1] - 1))
    iy = ((gy + np.float32(1.0)) * np.float32(0.5) * np.float32(_POLAR[0] - 1))
    ix0 = np.floor(ix)
    iy0 = np.floor(iy)
    wx0 = (np.float32(1.0) - (ix - ix0)).astype(np.float32)
    wy0 = (np.float32(1.0) - (iy - iy0)).astype(np.float32)
    i00 = (iy0.astype(np.int64) * _POLAR[1] + ix0.astype(np.int64)).astype(np.int32)
    box_idx = (_PLANE_W
               + (yy_org - _BOX_R0) * _BOX_W + (xx_org - _BOX_C0)).astype(np.int32)
    i00 = np.where(mask, i00, box_idx)
    wx0 = np.where(mask, wx0, np.float32(1.0)).astype(np.float32)
    wy0 = np.where(mask, wy0, np.float32(1.0)).astype(np.float32)
    return i00.reshape(-1), wx0.reshape(-1), wy0.reshape(-1)


_I00_NP, _WX0_NP, _WY0_NP = _build_static()


@functools.cache
def _make_sc_resample():
    mesh = plsc.VectorSubcoreMesh(core_axis_name="c", subcore_axis_name="s")
    return functools.partial(
        pl.kernel,
        mesh=mesh,
        out_type=jax.ShapeDtypeStruct((_PLANES, _NPX), jnp.float32),
        scratch_types=[
            pltpu.VMEM((_TW,), jnp.float32),
            pltpu.VMEM((_K,), jnp.int32),
            pltpu.VMEM((_K,), jnp.float32),
            pltpu.VMEM((_K,), jnp.float32),
            pltpu.VMEM((_K,), jnp.float32),
        ],
        compiler_params=pltpu.CompilerParams(needs_layout_passes=False),
    )(_sc_resample_body)


def _sc_resample_body(polar_hbm, refbox_hbm, i00_hbm, wx_hbm, wy_hbm, out_hbm,
                      table_v, idx_v, wx_v, wy_v, ob_v):
    wid = lax.axis_index("s") * 2 + lax.axis_index("c")

    zeros = jnp.zeros((16,), jnp.float32)

    def zbody(i, carry):
        table_v[pl.ds(_PLANE_W + _BOX + i * 16, 16)] = zeros
        return carry

    lax.fori_loop(0, _PAD // 16, zbody, 0)

    def plane_body(j, carry):
        p = wid * _PPT + j
        pltpu.sync_copy(polar_hbm.at[p], table_v.at[pl.ds(0, _PLANE_W)])
        pltpu.sync_copy(refbox_hbm.at[p], table_v.at[pl.ds(_PLANE_W, _BOX)])

        def chunk_body(cix, carry2):
            base = cix * _K
            pltpu.sync_copy(i00_hbm.at[pl.ds(base, _K)], idx_v)
            pltpu.sync_copy(wx_hbm.at[pl.ds(base, _K)], wx_v)
            pltpu.sync_copy(wy_hbm.at[pl.ds(base, _K)], wy_v)

            def px_body(i, carry3):
                o = i * 16
                idx = idx_v[pl.ds(o, 16)]
                wx0 = wx_v[pl.ds(o, 16)]
                wy0 = wy_v[pl.ds(o, 16)]
                v00 = plsc.load_gather(table_v, [idx])
                v01 = plsc.load_gather(table_v, [idx + 1])
                v10 = plsc.load_gather(table_v, [idx + _POLAR[1]])
                v11 = plsc.load_gather(table_v, [idx + (_POLAR[1] + 1)])
                wx1 = 1.0 - wx0
                wy1 = 1.0 - wy0
                r = wy0 * (wx0 * v00 + wx1 * v01) + wy1 * (wx0 * v10 + wx1 * v11)
                ob_v[pl.ds(o, 16)] = r
                return carry3

            lax.fori_loop(0, _K // 16, px_body, 0)
            pltpu.sync_copy(ob_v, out_hbm.at[p, pl.ds(base, _K)])
            return carry2

        lax.fori_loop(0, _NCHUNK, chunk_body, 0)
        return carry

    lax.fori_loop(0, _PPT, plane_body, 0)


def kernel(polar_feat, ref_feat):
    polar2d = polar_feat.reshape(_PLANES, _PLANE_W)
    refbox = ref_feat.reshape(_PLANES, _CART[0], _CART[1])[
        :, _BOX_R0:_BOX_R0 + _BOX_H, _BOX_C0:_BOX_C0 + _BOX_W
    ].reshape(_PLANES, _BOX)
    i00 = jnp.asarray(_I00_NP)
    wx0 = jnp.asarray(_WX0_NP)
    wy0 = jnp.asarray(_WY0_NP)
    out = _make_sc_resample()(polar2d, refbox, i00, wx0, wy0)
    return out.reshape(_B, _C, _CART[0], _CART[1])


# speedup vs baseline: 20.8046x; 1.3420x over previous
import functools

import jax
import jax.numpy as jnp
import numpy as np
from jax import lax
from jax.experimental import pallas as pl
from jax.experimental.pallas import tpu as pltpu
from jax.experimental.pallas import tpu_sc as plsc

_POLAR = (64, 1024)
_CART = (512, 512)
_CDGS = 3.0
_B, _C = 4, 32

_PLANES = _B * _C
_NPX = _CART[0] * _CART[1]
_PLANE_W = _POLAR[0] * _POLAR[1]
_BOX_R0, _BOX_C0, _BOX_H, _BOX_W = 240, 240, 32, 32
_BOX = _BOX_H * _BOX_W
_PAD = 1032
_TW = _PLANE_W + _BOX + _PAD

_K = 4096
_NCHUNK = _NPX // _K
_NTILES = 32
_PPT = _PLANES // _NTILES


def _build_static():
    yy_org, xx_org = np.meshgrid(np.arange(_CART[0]), np.arange(_CART[1]),
                                 indexing='ij')
    yy = (yy_org - _CART[0] / 2.0 + 0.5).astype(np.float32)
    xx = (xx_org - _CART[1] / 2.0 + 0.5).astype(np.float32)
    depth = np.sqrt(xx ** 2 + yy ** 2)
    phi = np.pi - np.arctan2(yy, xx)
    index_y = depth / (_CART[0] / 2.0 * np.sqrt(2.0)) * (_POLAR[0] + _CDGS) - _CDGS
    index_x = phi / np.pi / 2.0 * _POLAR[1]
    mask = index_y > 0
    gx = (index_x / _POLAR[1] * 2.0 - 1.0).astype(np.float32)
    gy = (-(index_y / _POLAR[0] * 2.0 - 1.0)).astype(np.float32)
    ix = ((gx + np.float32(1.0)) * np.float32(0.5) * np.float32(_POLAR[1] - 1))
    iy = ((gy + np.float32(1.0)) * np.float32(0.5) * np.float32(_POLAR[0] - 1))
    ix0 = np.floor(ix)
    iy0 = np.floor(iy)
    wx0 = (np.float32(1.0) - (ix - ix0)).astype(np.float32)
    wy0 = (np.float32(1.0) - (iy - iy0)).astype(np.float32)
    i00 = (iy0.astype(np.int64) * _POLAR[1] + ix0.astype(np.int64)).astype(np.int32)
    box_idx = (_PLANE_W
               + (yy_org - _BOX_R0) * _BOX_W + (xx_org - _BOX_C0)).astype(np.int32)
    i00 = np.where(mask, i00, box_idx)
    wx0 = np.where(mask, wx0, np.float32(1.0)).astype(np.float32)
    wy0 = np.where(mask, wy0, np.float32(1.0)).astype(np.float32)
    return i00.reshape(-1), wx0.reshape(-1), wy0.reshape(-1)


_I00_NP, _WX0_NP, _WY0_NP = _build_static()


@functools.cache
def _make_sc_resample():
    mesh = plsc.VectorSubcoreMesh(core_axis_name="c", subcore_axis_name="s")
    return functools.partial(
        pl.kernel,
        mesh=mesh,
        out_type=jax.ShapeDtypeStruct((_PLANES, _NPX), jnp.float32),
        scratch_types=[
            pltpu.VMEM((_TW,), jnp.float32),
            pltpu.VMEM((_K,), jnp.int32),
            pltpu.VMEM((_K,), jnp.float32),
            pltpu.VMEM((_K,), jnp.float32),
            pltpu.VMEM((_K,), jnp.float32),
        ],
        compiler_params=pltpu.CompilerParams(needs_layout_passes=False),
    )(_sc_resample_body)


def _sc_resample_body(polar_hbm, refbox_hbm, i00_hbm, wx_hbm, wy_hbm, out_hbm,
                      table_v, idx_v, wx_v, wy_v, ob_v):
    wid = lax.axis_index("s") * 2 + lax.axis_index("c")

    zeros = jnp.zeros((16,), jnp.float32)

    def zbody(i, carry):
        table_v[pl.ds(_PLANE_W + _BOX + i * 16, 16)] = zeros
        return carry

    lax.fori_loop(0, _PAD // 16, zbody, 0)

    def plane_body(j, carry):
        p = wid * _PPT + j
        pltpu.sync_copy(polar_hbm.at[p], table_v.at[pl.ds(0, _PLANE_W)])
        pltpu.sync_copy(refbox_hbm.at[p], table_v.at[pl.ds(_PLANE_W, _BOX)])

        def chunk_body(cix, carry2):
            base = cix * _K
            pltpu.sync_copy(i00_hbm.at[pl.ds(base, _K)], idx_v)
            pltpu.sync_copy(wx_hbm.at[pl.ds(base, _K)], wx_v)
            pltpu.sync_copy(wy_hbm.at[pl.ds(base, _K)], wy_v)

            @plsc.parallel_loop(0, _K, 16, unroll=8)
            def px_body(o):
                idx = idx_v[pl.ds(o, 16)]
                wx0 = wx_v[pl.ds(o, 16)]
                wy0 = wy_v[pl.ds(o, 16)]
                v00 = plsc.load_gather(table_v, [idx])
                v01 = plsc.load_gather(table_v, [idx + 1])
                v10 = plsc.load_gather(table_v, [idx + _POLAR[1]])
                v11 = plsc.load_gather(table_v, [idx + (_POLAR[1] + 1)])
                wx1 = 1.0 - wx0
                wy1 = 1.0 - wy0
                r = wy0 * (wx0 * v00 + wx1 * v01) + wy1 * (wx0 * v10 + wx1 * v11)
                ob_v[pl.ds(o, 16)] = r
            pltpu.sync_copy(ob_v, out_hbm.at[p, pl.ds(base, _K)])
            return carry2

        lax.fori_loop(0, _NCHUNK, chunk_body, 0)
        return carry

    lax.fori_loop(0, _PPT, plane_body, 0)


def kernel(polar_feat, ref_feat):
    polar2d = polar_feat.reshape(_PLANES, _PLANE_W)
    refbox = ref_feat.reshape(_PLANES, _CART[0], _CART[1])[
        :, _BOX_R0:_BOX_R0 + _BOX_H, _BOX_C0:_BOX_C0 + _BOX_W
    ].reshape(_PLANES, _BOX)
    i00 = jnp.asarray(_I00_NP)
    wx0 = jnp.asarray(_WX0_NP)
    wy0 = jnp.asarray(_WY0_NP)
    out = _make_sc_resample()(polar2d, refbox, i00, wx0, wy0)
    return out.reshape(_B, _C, _CART[0], _CART[1])


# speedup vs baseline: 32.3042x; 1.5527x over previous
import functools

import jax
import jax.numpy as jnp
import numpy as np
from jax import lax
from jax.experimental import pallas as pl
from jax.experimental.pallas import tpu as pltpu
from jax.experimental.pallas import tpu_sc as plsc

_POLAR = (64, 1024)
_CART = (512, 512)
_CDGS = 3.0
_B, _C = 4, 32

_PLANES = _B * _C
_QN = 256 * 256
_WW = 264
_WEDGE = _POLAR[0] * _WW
_BOXW = 16
_BOX = _BOXW * _BOXW
_PAD = 272
_TW = _WEDGE + _BOX + _PAD
_WC0 = (760, 508, 0, 256)

_K = 2048
_ROWS = _K // 256
_NCHUNK = _QN // _K
_PPASS = 4
_NPASS = 4


def _build_static():
    yy_org, xx_org = np.meshgrid(np.arange(_CART[0]), np.arange(_CART[1]),
                                 indexing='ij')
    yy = (yy_org - _CART[0] / 2.0 + 0.5).astype(np.float32)
    xx = (xx_org - _CART[1] / 2.0 + 0.5).astype(np.float32)
    depth = np.sqrt(xx ** 2 + yy ** 2)
    phi = np.pi - np.arctan2(yy, xx)
    index_y = depth / (_CART[0] / 2.0 * np.sqrt(2.0)) * (_POLAR[0] + _CDGS) - _CDGS
    index_x = phi / np.pi / 2.0 * _POLAR[1]
    mask = index_y > 0
    gx = (index_x / _POLAR[1] * 2.0 - 1.0).astype(np.float32)
    gy = (-(index_y / _POLAR[0] * 2.0 - 1.0)).astype(np.float32)
    ix = ((gx + np.float32(1.0)) * np.float32(0.5) * np.float32(_POLAR[1] - 1))
    iy = ((gy + np.float32(1.0)) * np.float32(0.5) * np.float32(_POLAR[0] - 1))
    ix0 = np.floor(ix)
    iy0 = np.floor(iy)
    wx0 = (np.float32(1.0) - (ix - ix0)).astype(np.float32)
    wy0 = (np.float32(1.0) - (iy - iy0)).astype(np.float32)

    i00_q, wx_q, wy_q = [], [], []
    for q in range(4):
        r0, c0 = (q // 2) * 256, (q % 2) * 256
        sl = (slice(r0, r0 + 256), slice(c0, c0 + 256))
        wix0 = ix0[sl] - _WC0[q]
        assert wix0[mask[sl]].min() >= 0 and wix0[mask[sl]].max() + 1 < _WW
        i00 = (iy0[sl].astype(np.int64) * _WW + wix0.astype(np.int64)).astype(np.int32)
        by0, bx0 = 240 if q < 2 else 256, 240 if q % 2 == 0 else 256
        box_idx = (_WEDGE + (yy_org[sl] - by0) * _BOXW
                   + (xx_org[sl] - bx0)).astype(np.int32)
        m = mask[sl]
        i00_q.append(np.where(m, i00, box_idx).reshape(-1))
        wx_q.append(np.where(m, wx0[sl], np.float32(1.0)).reshape(-1).astype(np.float32))
        wy_q.append(np.where(m, wy0[sl], np.float32(1.0)).reshape(-1).astype(np.float32))
    return (np.concatenate(i00_q), np.concatenate(wx_q), np.concatenate(wy_q))


_I00_NP, _WX0_NP, _WY0_NP = _build_static()


@functools.cache
def _make_sc_resample():
    mesh = plsc.VectorSubcoreMesh(core_axis_name="c", subcore_axis_name="s")
    return functools.partial(
        pl.kernel,
        mesh=mesh,
        out_type=jax.ShapeDtypeStruct((_PLANES, _CART[0], _CART[1]), jnp.float32),
        scratch_types=[
            [pltpu.VMEM((_TW,), jnp.float32) for _ in range(_PPASS)],
            pltpu.VMEM((_K,), jnp.int32),
            pltpu.VMEM((_K,), jnp.float32),
            pltpu.VMEM((_K,), jnp.float32),
            pltpu.VMEM((_ROWS, 256), jnp.float32),
        ],
        compiler_params=pltpu.CompilerParams(needs_layout_passes=False),
    )(_sc_resample_body)


def _sc_resample_body(wedges_hbm, boxes_hbm, i00_hbm, wx_hbm, wy_hbm, out_hbm,
                      tables_v, idx_v, wx_v, wy_v, ob_v):
    wid = lax.axis_index("s") * 2 + lax.axis_index("c")
    q = wid % 4
    grp = wid // 4
    qr0 = (q // 2) * 256
    qc0 = (q % 2) * 256

    zeros = jnp.zeros((16,), jnp.float32)
    for t in range(_PPASS):
        def zbody(i, carry, t=t):
            tables_v[t][pl.ds(_WEDGE + _BOX + i * 16, 16)] = zeros
            return carry
        lax.fori_loop(0, _PAD // 16, zbody, 0)

    def pass_body(ps, carry):
        pbase = grp * (_PPASS * _NPASS) + ps * _PPASS
        for t in range(_PPASS):
            pltpu.sync_copy(wedges_hbm.at[q, pbase + t],
                            tables_v[t].at[pl.ds(0, _WEDGE)])
            pltpu.sync_copy(boxes_hbm.at[q, pbase + t],
                            tables_v[t].at[pl.ds(_WEDGE, _BOX)])

        def chunk_body(cix, carry2):
            base = q * _QN + cix * _K
            pltpu.sync_copy(i00_hbm.at[pl.ds(base, _K)], idx_v)
            pltpu.sync_copy(wx_hbm.at[pl.ds(base, _K)], wx_v)
            pltpu.sync_copy(wy_hbm.at[pl.ds(base, _K)], wy_v)
            y0 = qr0 + cix * _ROWS
            for t in range(_PPASS):
                table_v = tables_v[t]

                @plsc.parallel_loop(0, _K, 16, unroll=8)
                def px_body(o):
                    idx = idx_v[pl.ds(o, 16)]
                    wx0 = wx_v[pl.ds(o, 16)]
                    wy0 = wy_v[pl.ds(o, 16)]
                    v00 = plsc.load_gather(table_v, [idx])
                    v01 = plsc.load_gather(table_v, [idx + 1])
                    v10 = plsc.load_gather(table_v, [idx + _WW])
                    v11 = plsc.load_gather(table_v, [idx + (_WW + 1)])
                    wx1 = 1.0 - wx0
                    wy1 = 1.0 - wy0
                    r = (wy0 * (wx0 * v00 + wx1 * v01)
                         + wy1 * (wx0 * v10 + wx1 * v11))
                    ob_v[o // 256, pl.ds(o % 256, 16)] = r

                pltpu.sync_copy(
                    ob_v,
                    out_hbm.at[pbase + t, pl.ds(y0, _ROWS), pl.ds(qc0, 256)])
            return carry2

        lax.fori_loop(0, _NCHUNK, chunk_body, 0)
        return carry

    lax.fori_loop(0, _NPASS, pass_body, 0)


def kernel(polar_feat, ref_feat):
    polar3 = polar_feat.reshape(_PLANES, _POLAR[0], _POLAR[1])
    ref3 = ref_feat.reshape(_PLANES, _CART[0], _CART[1])
    wedges = jnp.stack(
        [polar3[:, :, c0:c0 + _WW].reshape(_PLANES, _WEDGE) for c0 in _WC0])
    boxes = jnp.stack(
        [ref3[:, by0:by0 + _BOXW, bx0:bx0 + _BOXW].reshape(_PLANES, _BOX)
         for by0, bx0 in ((240, 240), (240, 256), (256, 240), (256, 256))])
    i00 = jnp.asarray(_I00_NP)
    wx0 = jnp.asarray(_WX0_NP)
    wy0 = jnp.asarray(_WY0_NP)
    out = _make_sc_resample()(wedges, boxes, i00, wx0, wy0)
    return out.reshape(_B, _C, _CART[0], _CART[1])


# speedup vs baseline: 54.4103x; 1.6843x over previous
import functools

import jax
import jax.numpy as jnp
import numpy as np
from jax import lax
from jax.experimental import pallas as pl
from jax.experimental.pallas import tpu as pltpu
from jax.experimental.pallas import tpu_sc as plsc

_POLAR = (64, 1024)
_CART = (512, 512)
_CDGS = 3.0
_B, _C = 4, 32

_PLANES = _B * _C
_QN = 256 * 256
_WW = 264
_WEDGE = _POLAR[0] * _WW
_BOXW = 16
_BOX = _BOXW * _BOXW
_PAD = 272
_TW = _WEDGE + _BOX + _PAD
_WC0 = (760, 508, 0, 256)

_K = 2048
_ROWS = _K // 256
_NCHUNK = _QN // _K
_PPASS = 4
_NPASS = 4


def _build_static():
    yy_org, xx_org = np.meshgrid(np.arange(_CART[0]), np.arange(_CART[1]),
                                 indexing='ij')
    yy = (yy_org - _CART[0] / 2.0 + 0.5).astype(np.float32)
    xx = (xx_org - _CART[1] / 2.0 + 0.5).astype(np.float32)
    depth = np.sqrt(xx ** 2 + yy ** 2)
    phi = np.pi - np.arctan2(yy, xx)
    index_y = depth / (_CART[0] / 2.0 * np.sqrt(2.0)) * (_POLAR[0] + _CDGS) - _CDGS
    index_x = phi / np.pi / 2.0 * _POLAR[1]
    mask = index_y > 0
    gx = (index_x / _POLAR[1] * 2.0 - 1.0).astype(np.float32)
    gy = (-(index_y / _POLAR[0] * 2.0 - 1.0)).astype(np.float32)
    ix = ((gx + np.float32(1.0)) * np.float32(0.5) * np.float32(_POLAR[1] - 1))
    iy = ((gy + np.float32(1.0)) * np.float32(0.5) * np.float32(_POLAR[0] - 1))
    ix0 = np.floor(ix)
    iy0 = np.floor(iy)
    wx0 = (np.float32(1.0) - (ix - ix0)).astype(np.float32)
    wy0 = (np.float32(1.0) - (iy - iy0)).astype(np.float32)

    stream = np.empty((4, _QN // _K, 3, _K), np.int32)
    for q in range(4):
        r0, c0 = (q // 2) * 256, (q % 2) * 256
        sl = (slice(r0, r0 + 256), slice(c0, c0 + 256))
        wix0 = ix0[sl] - _WC0[q]
        assert wix0[mask[sl]].min() >= 0 and wix0[mask[sl]].max() + 1 < _WW
        i00 = (iy0[sl].astype(np.int64) * _WW + wix0.astype(np.int64)).astype(np.int32)
        by0, bx0 = 240 if q < 2 else 256, 240 if q % 2 == 0 else 256
        box_idx = (_WEDGE + (yy_org[sl] - by0) * _BOXW
                   + (xx_org[sl] - bx0)).astype(np.int32)
        m = mask[sl]
        iq = np.where(m, i00, box_idx).reshape(-1)
        wxq = np.where(m, wx0[sl], np.float32(1.0)).reshape(-1).astype(np.float32)
        wyq = np.where(m, wy0[sl], np.float32(1.0)).reshape(-1).astype(np.float32)
        stream[q, :, 0, :] = iq.reshape(-1, _K)
        stream[q, :, 1, :] = wxq.view(np.int32).reshape(-1, _K)
        stream[q, :, 2, :] = wyq.view(np.int32).reshape(-1, _K)
    return stream.reshape(4 * (_QN // _K), 3 * _K)


_STREAM_NP = _build_static()


@functools.cache
def _make_sc_resample():
    mesh = plsc.VectorSubcoreMesh(core_axis_name="c", subcore_axis_name="s")
    return functools.partial(
        pl.kernel,
        mesh=mesh,
        out_type=jax.ShapeDtypeStruct((_PLANES, _CART[0], _CART[1]), jnp.float32),
        scratch_types=[
            [pltpu.VMEM((_TW,), jnp.float32) for _ in range(_PPASS)],
            [pltpu.VMEM((3 * _K,), jnp.int32) for _ in range(2)],
            [[pltpu.VMEM((_ROWS, 256), jnp.float32) for _ in range(2)]
             for _ in range(_PPASS)],
            [pltpu.SemaphoreType.DMA for _ in range(2)],
            [pltpu.SemaphoreType.DMA for _ in range(2)],
        ],
        compiler_params=pltpu.CompilerParams(needs_layout_passes=False),
    )(_sc_resample_body)


def _sc_resample_body(wedges_hbm, boxes_hbm, stream_hbm, out_hbm,
                      tables_v, inbufs_v, obufs_v, sem_in, sem_out):
    wid = lax.axis_index("s") * 2 + lax.axis_index("c")
    q = wid % 4
    grp = wid // 4
    qr0 = (q // 2) * 256
    qc0 = (q % 2) * 256

    zeros = jnp.zeros((16,), jnp.float32)
    for t in range(_PPASS):
        def zbody(i, carry, t=t):
            tables_v[t][pl.ds(_WEDGE + _BOX + i * 16, 16)] = zeros
            return carry
        lax.fori_loop(0, _PAD // 16, zbody, 0)

    def start_in(cix, b):
        pltpu.async_copy(stream_hbm.at[q * _NCHUNK + cix], inbufs_v[b],
                         sem_in[b])

    def wait_in(b):
        pltpu.make_async_copy(stream_hbm.at[0], inbufs_v[b], sem_in[b]).wait()

    def drain_out(b):
        for t in range(_PPASS):
            pltpu.make_async_copy(
                out_hbm.at[0, pl.ds(0, _ROWS), pl.ds(0, 256)],
                obufs_v[t][b], sem_out[b]).wait()

    def pass_body(ps, carry):
        pbase = grp * (_PPASS * _NPASS) + ps * _PPASS
        for t in range(_PPASS):
            pltpu.sync_copy(wedges_hbm.at[q, pbase + t],
                            tables_v[t].at[pl.ds(0, _WEDGE)])
            pltpu.sync_copy(boxes_hbm.at[q, pbase + t],
                            tables_v[t].at[pl.ds(_WEDGE, _BOX)])

        start_in(0, 0)
        start_in(1, 1)

        def chunk_pair_body(half, carry2):
            for b in range(2):
                cix = half * 2 + b
                wait_in(b)
                pl.when(cix >= 2)(lambda b=b: drain_out(b))
                inbuf = inbufs_v[b]
                y0 = qr0 + cix * _ROWS
                for t in range(_PPASS):
                    table_v = tables_v[t]
                    ob_v = obufs_v[t][b]

                    @plsc.parallel_loop(0, _K, 16, unroll=8)
                    def px_body(o):
                        idx = inbuf[pl.ds(o, 16)]
                        wx0 = plsc.bitcast(inbuf[pl.ds(_K + o, 16)], jnp.float32)
                        wy0 = plsc.bitcast(inbuf[pl.ds(2 * _K + o, 16)], jnp.float32)
                        v00 = plsc.load_gather(table_v, [idx])
                        v01 = plsc.load_gather(table_v, [idx + 1])
                        v10 = plsc.load_gather(table_v, [idx + _WW])
                        v11 = plsc.load_gather(table_v, [idx + (_WW + 1)])
                        wx1 = 1.0 - wx0
                        wy1 = 1.0 - wy0
                        r = (wy0 * (wx0 * v00 + wx1 * v01)
                             + wy1 * (wx0 * v10 + wx1 * v11))
                        ob_v[o // 256, pl.ds(o % 256, 16)] = r

                    pltpu.async_copy(
                        ob_v,
                        out_hbm.at[pbase + t, pl.ds(y0, _ROWS), pl.ds(qc0, 256)],
                        sem_out[b])
                pl.when(cix + 2 < _NCHUNK)(lambda cix=cix, b=b: start_in(cix + 2, b))
            return carry2

        lax.fori_loop(0, _NCHUNK // 2, chunk_pair_body, 0)
        drain_out(0)
        drain_out(1)
        return carry

    lax.fori_loop(0, _NPASS, pass_body, 0)


def kernel(polar_feat, ref_feat):
    polar3 = polar_feat.reshape(_PLANES, _POLAR[0], _POLAR[1])
    ref3 = ref_feat.reshape(_PLANES, _CART[0], _CART[1])
    wedges = jnp.stack(
        [polar3[:, :, c0:c0 + _WW].reshape(_PLANES, _WEDGE) for c0 in _WC0])
    boxes = jnp.stack(
        [ref3[:, by0:by0 + _BOXW, bx0:bx0 + _BOXW].reshape(_PLANES, _BOX)
         for by0, bx0 in ((240, 240), (240, 256), (256, 240), (256, 256))])
    stream = jnp.asarray(_STREAM_NP)
    out = _make_sc_resample()(wedges, boxes, stream)
    return out.reshape(_B, _C, _CART[0], _CART[1])
